# baseline (speedup 1.0000x reference)
import functools

import jax
import jax.numpy as jnp
from jax import lax
from jax.experimental import pallas as pl
from jax.experimental.pallas import tpu as pltpu
from jax.experimental.pallas import tpu_sc as plsc

_B, _C, _H, _W = 8, 192, 224, 224
_ROWS = _B * _C
_D = _H * _W
_NC = 2
_RPC = _ROWS // _NC
_G = 16
_NGRP = _RPC // _G

_mesh = plsc.ScalarSubcoreMesh(axis_name="c", num_cores=_NC)


@functools.partial(
    pl.kernel,
    mesh=_mesh,
    out_type=jax.ShapeDtypeStruct((_ROWS, _D), jnp.float32),
    scratch_types=[
        pltpu.VMEM_SHARED((2, _G, _D), jnp.float32),
        pltpu.SemaphoreType.DMA((2,)),
        pltpu.SemaphoreType.DMA((2,)),
    ],
)
def _reverse_rows_scs(in_hbm, out_hbm, bufs, in_sems, out_sems):
    cid = lax.axis_index("c")
    row0 = cid * _RPC

    def grp_info(g):
        r0 = row0 + g * _G
        b = r0 // _C
        c0 = lax.rem(r0, _C)
        s0 = b * _C + (_C - 1 - c0 - (_G - 1))
        return r0, s0

    def start_in(g):
        slot = lax.rem(g, 2)
        _, s0 = grp_info(g)
        pltpu.async_copy(in_hbm.at[pl.ds(s0, _G)], bufs.at[slot],
                         in_sems.at[slot])

    def wait_in(g):
        slot = lax.rem(g, 2)
        _, s0 = grp_info(g)
        pltpu.make_async_copy(in_hbm.at[pl.ds(s0, _G)], bufs.at[slot],
                              in_sems.at[slot]).wait()

    def start_outs(g):
        slot = lax.rem(g, 2)
        r0, _ = grp_info(g)
        for j in range(_G):
            pltpu.async_copy(bufs.at[slot, _G - 1 - j], out_hbm.at[r0 + j],
                             out_sems.at[slot])

    def wait_outs(g):
        slot = lax.rem(g, 2)
        r0, _ = grp_info(g)
        for j in range(_G):
            pltpu.make_async_copy(bufs.at[slot, _G - 1 - j],
                                  out_hbm.at[r0 + j],
                                  out_sems.at[slot]).wait()

    start_in(0)

    def body(g, carry):
        pl.when(g + 1 < _NGRP)(lambda: start_in(g + 1))
        wait_in(g)
        pl.when(g >= 2)(lambda: wait_outs(g - 2))
        start_outs(g)
        return carry

    lax.fori_loop(0, _NGRP, body, 0)
    wait_outs(_NGRP - 2)
    wait_outs(_NGRP - 1)


def kernel(input):
    x = input.reshape(_ROWS, _D)
    y = _reverse_rows_scs(x)
    return y.reshape(_B, _C, _H, _W)
